# Initial kernel scaffold; baseline (speedup 1.0000x reference)
import jax
import jax.numpy as jnp
from jax.experimental import pallas as pl


def kernel(u, v, e_indices, e_values, Wg1, bg1, Wg2, bg2, Wf1, bf1):
    raise NotImplementedError("write your pallas kernel here")


# speedup vs baseline: 1.5868x; 1.5868x over previous
import functools

import jax
import jax.numpy as jnp
from jax import lax
from jax.experimental import pallas as pl
from jax.experimental.pallas import tpu as pltpu
from jax.experimental.pallas import tpu_sc as plsc

_U, _V, _E = 10000, 10000, 160000
_F, _G, _H, _D = 256, 256, 16, 256
_HID = 512

_NC, _NS, _L = 2, 16, 16
_NW = _NC * _NS

_EPW = _E // _NW
_KG = 40
_NBG = _EPW // _KG

_CH = _D // _NC
_EPS = _E // _NS
_K2 = 80
_NB2 = _EPS // _K2
_ACC = 10240
_RPS = _ACC // _NS
_TAIL = _U - (_NS - 1) * _RPS


def _sc_mesh():
    return plsc.VectorSubcoreMesh(
        core_axis_name="c", subcore_axis_name="s", num_cores=_NC, num_subcores=_NS
    )


def _pre_body(u_ref, v_ref, wu_ref, wv_ref, b1_ref, a_ref, b_ref):
    a_ref[...] = (
        jnp.dot(u_ref[...], wu_ref[...], preferred_element_type=jnp.float32)
        + b1_ref[...]
    )
    b_ref[...] = jnp.dot(v_ref[...], wv_ref[...], preferred_element_type=jnp.float32)


def _precompute(u, v, wu, wv, b1):
    rb = 1000
    return pl.pallas_call(
        _pre_body,
        grid=(_U // rb,),
        in_specs=[
            pl.BlockSpec((rb, _F), lambda i: (i, 0)),
            pl.BlockSpec((rb, _G), lambda i: (i, 0)),
            pl.BlockSpec((_F, _HID), lambda i: (0, 0)),
            pl.BlockSpec((_G, _HID), lambda i: (0, 0)),
            pl.BlockSpec((1, _HID), lambda i: (0, 0)),
        ],
        out_specs=[
            pl.BlockSpec((rb, _HID), lambda i: (i, 0)),
            pl.BlockSpec((rb, _HID), lambda i: (i, 0)),
        ],
        out_shape=[
            jax.ShapeDtypeStruct((_U, _HID), jnp.float32),
            jax.ShapeDtypeStruct((_V, _HID), jnp.float32),
        ],
    )(u, v, wu, wv, b1)


def _gather_body(a_hbm, b_hbm, ui_hbm, vi_hbm, s_hbm, idxu, idxv, ra, rb, sema, semb):
    wid = lax.axis_index("s") * _NC + lax.axis_index("c")
    base = wid * _EPW

    def blk(j, carry):
        off = pl.multiple_of(base + j * _KG, _KG)
        pltpu.sync_copy(ui_hbm.at[pl.ds(off, _KG)], idxu)
        pltpu.sync_copy(vi_hbm.at[pl.ds(off, _KG)], idxv)
        ca = pltpu.async_copy(a_hbm.at[idxu], ra, sema)
        cb = pltpu.async_copy(b_hbm.at[idxv], rb, semb)
        ca.wait()
        cb.wait()

        def addrow(r, c2):
            for c in range(_HID // _L):
                sl = pl.ds(c * _L, _L)
                ra[r, sl] = ra[r, sl] + rb[r, sl]
            return c2

        lax.fori_loop(0, _KG, addrow, 0)
        pltpu.sync_copy(ra, s_hbm.at[pl.ds(off, _KG)])
        return carry

    lax.fori_loop(0, _NBG, blk, 0)


def _gather_add(a, b, ui, vi):
    fn = pl.kernel(
        _gather_body,
        out_type=jax.ShapeDtypeStruct((_E, _HID), jnp.float32),
        mesh=_sc_mesh(),
        scratch_types=[
            pltpu.VMEM((_KG,), jnp.int32),
            pltpu.VMEM((_KG,), jnp.int32),
            pltpu.VMEM((_KG, _HID), jnp.float32),
            pltpu.VMEM((_KG, _HID), jnp.float32),
            pltpu.SemaphoreType.DMA,
            pltpu.SemaphoreType.DMA,
        ],
    )
    return fn(a, b, ui, vi)


def _mlp_body(s_ref, ev_ref, we_ref, w2_ref, b2_ref, g_ref):
    h = s_ref[...] + jnp.dot(
        ev_ref[...], we_ref[...], preferred_element_type=jnp.float32
    )
    h = jnp.maximum(h, 0.0)
    g = (
        jnp.dot(h, w2_ref[...], preferred_element_type=jnp.float32)
        + b2_ref[...]
    )
    g_ref[...] = jnp.maximum(g, 0.0)


def _edge_mlp(s, ev, we, w2, b2):
    be = 1280
    return pl.pallas_call(
        _mlp_body,
        grid=(_E // be,),
        in_specs=[
            pl.BlockSpec((be, _HID), lambda i: (i, 0)),
            pl.BlockSpec((be, _H), lambda i: (i, 0)),
            pl.BlockSpec((_H, _HID), lambda i: (0, 0)),
            pl.BlockSpec((_HID, _D), lambda i: (0, 0)),
            pl.BlockSpec((1, _D), lambda i: (0, 0)),
        ],
        out_specs=pl.BlockSpec((be, _D), lambda i: (i, 0)),
        out_shape=jax.ShapeDtypeStruct((_E, _D), jnp.float32),
    )(s, ev, we, w2, b2)


def _scatter_body(g_hbm, ui_hbm, out_hbm, idx, rows, acc):
    cid = lax.axis_index("c")
    sid = lax.axis_index("s")
    col = pl.multiple_of(cid * _CH, _CH)
    rbase = pl.multiple_of(sid * _RPS, _RPS)
    zero = jnp.zeros((_L,), jnp.float32)

    def zrow(r, carry):
        for c in range(_CH // _L):
            rows[r, pl.ds(c * _L, _L)] = zero
        return carry

    lax.fori_loop(0, _K2, zrow, 0)
    for k in range(_RPS // _K2):
        pltpu.sync_copy(rows, acc.at[pl.ds(rbase + k * _K2, _K2)])
    plsc.subcore_barrier()

    def blk(j, carry):
        off = pl.multiple_of(sid * _EPS + j * _K2, _K2)
        pltpu.sync_copy(ui_hbm.at[pl.ds(off, _K2)], idx)
        pltpu.sync_copy(g_hbm.at[pl.ds(off, _K2), pl.ds(col, _CH)], rows)
        pltpu.sync_copy(rows, acc.at[idx], add=True)
        return carry

    lax.fori_loop(0, _NB2, blk, 0)
    plsc.subcore_barrier()

    @pl.when(sid < _NS - 1)
    def _copy_full():
        pltpu.sync_copy(
            acc.at[pl.ds(rbase, _RPS)], out_hbm.at[pl.ds(rbase, _RPS), pl.ds(col, _CH)]
        )

    @pl.when(sid == _NS - 1)
    def _copy_tail():
        tb = (_NS - 1) * _RPS
        pltpu.sync_copy(
            acc.at[pl.ds(tb, _TAIL)], out_hbm.at[pl.ds(tb, _TAIL), pl.ds(col, _CH)]
        )


def _segment_sum(g, ui):
    fn = pl.kernel(
        _scatter_body,
        out_type=jax.ShapeDtypeStruct((_U, _D), jnp.float32),
        mesh=_sc_mesh(),
        scratch_types=[
            pltpu.VMEM((_K2,), jnp.int32),
            pltpu.VMEM((_K2, _CH), jnp.float32),
            pltpu.VMEM_SHARED((_ACC, _CH), jnp.float32),
        ],
    )
    return fn(g, ui)


def _fin_body(u_ref, agg_ref, wu_ref, wa_ref, b_ref, o_ref):
    o = (
        jnp.dot(u_ref[...], wu_ref[...], preferred_element_type=jnp.float32)
        + jnp.dot(agg_ref[...], wa_ref[...], preferred_element_type=jnp.float32)
        + b_ref[...]
    )
    o_ref[...] = jnp.maximum(o, 0.0)


def _final(u, agg, wu, wa, b):
    rb = 1000
    return pl.pallas_call(
        _fin_body,
        grid=(_U // rb,),
        in_specs=[
            pl.BlockSpec((rb, _F), lambda i: (i, 0)),
            pl.BlockSpec((rb, _D), lambda i: (i, 0)),
            pl.BlockSpec((_F, _D), lambda i: (0, 0)),
            pl.BlockSpec((_D, _D), lambda i: (0, 0)),
            pl.BlockSpec((1, _D), lambda i: (0, 0)),
        ],
        out_specs=pl.BlockSpec((rb, _D), lambda i: (i, 0)),
        out_shape=jax.ShapeDtypeStruct((_U, _D), jnp.float32),
    )(u, agg, wu, wa, b)


def kernel(u, v, e_indices, e_values, Wg1, bg1, Wg2, bg2, Wf1, bf1):
    vi = e_indices[0]
    ui = e_indices[1]
    wu = Wg1[:_F]
    wv = Wg1[_F : _F + _G]
    we = Wg1[_F + _G :]
    a, b = _precompute(u, v, wu, wv, bg1.reshape(1, _HID))
    s = _gather_add(a, b, ui, vi)
    g = _edge_mlp(s, e_values, we, Wg2, bg2.reshape(1, _D))
    agg = _segment_sum(g, ui)
    return _final(u, agg, Wf1[:_F], Wf1[_F:], bf1.reshape(1, _D))
